# initial kernel scaffold (unmeasured)
import jax
import jax.numpy as jnp
from jax import lax
from jax.experimental import pallas as pl
from jax.experimental.pallas import tpu as pltpu

N_DEV = 8
SQ = 2048
SKV = 2048
HQ_LOCAL = 8
DH = 128
DM = 1024
SCALE = 0.08838834764831843
QBLK = 512
NQ = SQ // QBLK
BLK = 64


def _attn_body(x_ref, wq_ref, k_ref, v_ref, wo_ref, out_ref):
    qi = pl.program_id(0)
    h = pl.program_id(1)

    xb = x_ref[...]
    wq = wq_ref[...].astype(jnp.bfloat16)
    q = lax.dot_general(xb, wq, (((1,), (0,)), ((), ())),
                        preferred_element_type=jnp.float32)
    qs = (q * SCALE).astype(jnp.bfloat16)

    kb = k_ref[...][:, 0, :].astype(jnp.bfloat16)
    s = lax.dot_general(qs, kb, (((1,), (1,)), ((), ())),
                        preferred_element_type=jnp.float32)

    rowb = (qi * QBLK + lax.broadcasted_iota(jnp.int32, (QBLK, SKV), 0)) // BLK
    colb = lax.broadcasted_iota(jnp.int32, (QBLK, SKV), 1) // BLK
    s = jnp.where(colb <= rowb, s, -1e9)

    m = jnp.max(s, axis=1, keepdims=True)
    w = jnp.exp(s - m)
    denom = jnp.sum(w, axis=1, keepdims=True)
    wb = (w / denom).astype(jnp.bfloat16)

    vb = v_ref[...][:, 0, :].astype(jnp.bfloat16)
    ctx = lax.dot_general(wb, vb, (((1,), (0,)), ((), ())),
                          preferred_element_type=jnp.float32)
    ctxb = ctx.astype(jnp.bfloat16)

    wo = wo_ref[...].astype(jnp.bfloat16)
    p = lax.dot_general(ctxb, wo, (((1,), (0,)), ((), ())),
                        preferred_element_type=jnp.float32)

    @pl.when(h == 0)
    def _():
        out_ref[...] = p

    @pl.when(h != 0)
    def _():
        out_ref[...] += p


def _ring_body(p_ref, out_ref, comm_ref, send_sems, recv_sems, credit_sem):
    my = lax.axis_index("i")
    left = lax.rem(my - 1 + N_DEV, N_DEV)
    right = lax.rem(my + 1, N_DEV)

    barrier_sem = pltpu.get_barrier_semaphore()
    for nbr in (left, right):
        pl.semaphore_signal(barrier_sem, inc=1, device_id=(nbr,),
                            device_id_type=pl.DeviceIdType.MESH)
    pl.semaphore_wait(barrier_sem, 2)

    out_ref[...] = p_ref[...]
    comm_ref[0, :, :] = p_ref[...]

    for h in range(N_DEV - 1):
        send_slot = h % 2
        recv_slot = (h + 1) % 2
        if h >= 1:
            pl.semaphore_wait(credit_sem, 1)
        rdma = pltpu.make_async_remote_copy(
            src_ref=comm_ref.at[send_slot],
            dst_ref=comm_ref.at[recv_slot],
            send_sem=send_sems.at[h],
            recv_sem=recv_sems.at[h],
            device_id=(right,),
            device_id_type=pl.DeviceIdType.MESH,
        )
        rdma.start()
        rdma.wait()
        out_ref[...] += comm_ref[recv_slot, :, :]
        if h < N_DEV - 2:
            pl.semaphore_signal(credit_sem, inc=1, device_id=(left,),
                                device_id_type=pl.DeviceIdType.MESH)


def kernel(x, Wq, K_ext, V_ext, Wo):
    i = lax.axis_index("i")

    x2 = x.reshape(SQ, DM).astype(jnp.bfloat16)
    K = K_ext.reshape(SKV, HQ_LOCAL, DH)
    V = V_ext.reshape(SKV, HQ_LOCAL, DH)
    Wq_i = lax.dynamic_slice_in_dim(Wq, i * (HQ_LOCAL * DH), HQ_LOCAL * DH, 1)
    Wo_i = lax.dynamic_slice_in_dim(Wo, i * (HQ_LOCAL * DH), HQ_LOCAL * DH, 0)

    partial = pl.pallas_call(
        _attn_body,
        grid=(NQ, HQ_LOCAL),
        in_specs=[
            pl.BlockSpec((QBLK, DM), lambda qi, h: (qi, 0)),
            pl.BlockSpec((DM, DH), lambda qi, h: (0, h)),
            pl.BlockSpec((SKV, 1, DH), lambda qi, h: (0, h, 0)),
            pl.BlockSpec((SKV, 1, DH), lambda qi, h: (0, h, 0)),
            pl.BlockSpec((DH, DM), lambda qi, h: (h, 0)),
        ],
        out_specs=pl.BlockSpec((QBLK, DM), lambda qi, h: (qi, 0)),
        out_shape=jax.ShapeDtypeStruct((SQ, DM), jnp.float32),
    )(x2, Wq_i, K, V, Wo_i)

    out = pl.pallas_call(
        _ring_body,
        out_shape=jax.ShapeDtypeStruct((SQ, DM), jnp.float32),
        in_specs=[pl.BlockSpec(memory_space=pltpu.VMEM)],
        out_specs=pl.BlockSpec(memory_space=pltpu.VMEM),
        scratch_shapes=[
            pltpu.VMEM((2, SQ, DM), jnp.float32),
            pltpu.SemaphoreType.DMA((N_DEV - 1,)),
            pltpu.SemaphoreType.DMA((N_DEV - 1,)),
            pltpu.SemaphoreType.REGULAR,
        ],
        compiler_params=pltpu.CompilerParams(collective_id=0),
    )(partial)

    return out.reshape(1, SQ, DM)


# baseline (device time: 818430 ns/iter reference)
import jax
import jax.numpy as jnp
from jax import lax
from jax.experimental import pallas as pl
from jax.experimental.pallas import tpu as pltpu

N_DEV = 8
SQ = 2048
SKV = 2048
HQ_LOCAL = 8
DH = 128
DM = 1024
SCALE = 0.08838834764831843
QBLK = 512
NQ = SQ // QBLK
BLK = 64


def _attn_body(x_ref, wq_ref, k_ref, v_ref, wo_ref, out_ref):
    qi = pl.program_id(0)
    h = pl.program_id(1)

    xb = x_ref[...]
    wq = wq_ref[...].astype(jnp.bfloat16)
    q = lax.dot_general(xb, wq, (((1,), (0,)), ((), ())),
                        preferred_element_type=jnp.float32)
    qs = (q * SCALE).astype(jnp.bfloat16)

    kb = k_ref[...][0, :, :].astype(jnp.bfloat16)
    s = lax.dot_general(qs, kb, (((1,), (1,)), ((), ())),
                        preferred_element_type=jnp.float32)

    rowb = (qi * QBLK + lax.broadcasted_iota(jnp.int32, (QBLK, SKV), 0)) // BLK
    colb = lax.broadcasted_iota(jnp.int32, (QBLK, SKV), 1) // BLK
    s = jnp.where(colb <= rowb, s, -1e9)

    m = jnp.max(s, axis=1, keepdims=True)
    w = jnp.exp(s - m)
    denom = jnp.sum(w, axis=1, keepdims=True)
    wb = (w / denom).astype(jnp.bfloat16)

    vb = v_ref[...][0, :, :].astype(jnp.bfloat16)
    ctx = lax.dot_general(wb, vb, (((1,), (0,)), ((), ())),
                          preferred_element_type=jnp.float32)
    ctxb = ctx.astype(jnp.bfloat16)

    wo = wo_ref[...].astype(jnp.bfloat16)
    p = lax.dot_general(ctxb, wo, (((1,), (0,)), ((), ())),
                        preferred_element_type=jnp.float32)

    @pl.when(h == 0)
    def _():
        out_ref[...] = p

    @pl.when(h != 0)
    def _():
        out_ref[...] += p


def _ring_body(p_ref, out_ref, comm_ref, send_sems, recv_sems, credit_sem):
    my = lax.axis_index("i")
    left = lax.rem(my - 1 + N_DEV, N_DEV)
    right = lax.rem(my + 1, N_DEV)

    barrier_sem = pltpu.get_barrier_semaphore()
    for nbr in (left, right):
        pl.semaphore_signal(barrier_sem, inc=1, device_id=(nbr,),
                            device_id_type=pl.DeviceIdType.MESH)
    pl.semaphore_wait(barrier_sem, 2)

    out_ref[...] = p_ref[...]
    comm_ref[0, :, :] = p_ref[...]

    for h in range(N_DEV - 1):
        send_slot = h % 2
        recv_slot = (h + 1) % 2
        if h >= 1:
            pl.semaphore_wait(credit_sem, 1)
        rdma = pltpu.make_async_remote_copy(
            src_ref=comm_ref.at[send_slot],
            dst_ref=comm_ref.at[recv_slot],
            send_sem=send_sems.at[h],
            recv_sem=recv_sems.at[h],
            device_id=(right,),
            device_id_type=pl.DeviceIdType.MESH,
        )
        rdma.start()
        rdma.wait()
        out_ref[...] += comm_ref[recv_slot, :, :]
        if h < N_DEV - 2:
            pl.semaphore_signal(credit_sem, inc=1, device_id=(left,),
                                device_id_type=pl.DeviceIdType.MESH)


def kernel(x, Wq, K_ext, V_ext, Wo):
    i = lax.axis_index("i")

    x2 = x.reshape(SQ, DM).astype(jnp.bfloat16)
    K = K_ext.reshape(SKV, HQ_LOCAL, DH).transpose(1, 0, 2)
    V = V_ext.reshape(SKV, HQ_LOCAL, DH).transpose(1, 0, 2)
    Wq_i = lax.dynamic_slice_in_dim(Wq, i * (HQ_LOCAL * DH), HQ_LOCAL * DH, 1)
    Wo_i = lax.dynamic_slice_in_dim(Wo, i * (HQ_LOCAL * DH), HQ_LOCAL * DH, 0)

    partial = pl.pallas_call(
        _attn_body,
        grid=(NQ, HQ_LOCAL),
        in_specs=[
            pl.BlockSpec((QBLK, DM), lambda qi, h: (qi, 0)),
            pl.BlockSpec((DM, DH), lambda qi, h: (0, h)),
            pl.BlockSpec((1, SKV, DH), lambda qi, h: (h, 0, 0)),
            pl.BlockSpec((1, SKV, DH), lambda qi, h: (h, 0, 0)),
            pl.BlockSpec((DH, DM), lambda qi, h: (h, 0)),
        ],
        out_specs=pl.BlockSpec((QBLK, DM), lambda qi, h: (qi, 0)),
        out_shape=jax.ShapeDtypeStruct((SQ, DM), jnp.float32),
    )(x2, Wq_i, K, V, Wo_i)

    out = pl.pallas_call(
        _ring_body,
        out_shape=jax.ShapeDtypeStruct((SQ, DM), jnp.float32),
        in_specs=[pl.BlockSpec(memory_space=pltpu.VMEM)],
        out_specs=pl.BlockSpec(memory_space=pltpu.VMEM),
        scratch_shapes=[
            pltpu.VMEM((2, SQ, DM), jnp.float32),
            pltpu.SemaphoreType.DMA((N_DEV - 1,)),
            pltpu.SemaphoreType.DMA((N_DEV - 1,)),
            pltpu.SemaphoreType.REGULAR,
        ],
        compiler_params=pltpu.CompilerParams(collective_id=0),
    )(partial)

    return out.reshape(1, SQ, DM)


# device time: 342830 ns/iter; 2.3873x vs baseline; 2.3873x over previous
import jax
import jax.numpy as jnp
from jax import lax
from jax.experimental import pallas as pl
from jax.experimental.pallas import tpu as pltpu

N_DEV = 8
SQ = 2048
SKV = 2048
HQ_LOCAL = 8
DH = 128
DM = 1024
SCALE = 0.08838834764831843
QBLK = 512
NQ = SQ // QBLK
BLK = 64


def _attn_body(x_ref, wq_ref, k_ref, v_ref, wo_ref, out_ref):
    qi = pl.program_id(0)
    h = pl.program_id(1)

    xb = x_ref[...]
    wq = wq_ref[...].astype(jnp.bfloat16)
    q = lax.dot_general(xb, wq, (((1,), (0,)), ((), ())),
                        preferred_element_type=jnp.float32)
    qs = (q * SCALE).astype(jnp.bfloat16)

    kb = k_ref[...][0, :, :].astype(jnp.bfloat16)
    s = lax.dot_general(qs, kb, (((1,), (1,)), ((), ())),
                        preferred_element_type=jnp.float32)

    rowb = (qi * QBLK + lax.broadcasted_iota(jnp.int32, (QBLK, SKV), 0)) // BLK
    colb = lax.broadcasted_iota(jnp.int32, (QBLK, SKV), 1) // BLK
    s = jnp.where(colb <= rowb, s, -1e9)

    m = jnp.max(s, axis=1, keepdims=True)
    w = jnp.exp(s - m)
    denom = jnp.sum(w, axis=1, keepdims=True)
    wb = (w / denom).astype(jnp.bfloat16)

    vb = v_ref[...][0, :, :].astype(jnp.bfloat16)
    ctx = lax.dot_general(wb, vb, (((1,), (0,)), ((), ())),
                          preferred_element_type=jnp.float32)
    ctxb = ctx.astype(jnp.bfloat16)

    wo = wo_ref[...].astype(jnp.bfloat16)
    p = lax.dot_general(ctxb, wo, (((1,), (0,)), ((), ())),
                        preferred_element_type=jnp.float32)

    @pl.when(h == 0)
    def _():
        out_ref[...] = p

    @pl.when(h != 0)
    def _():
        out_ref[...] += p


CH = SQ // N_DEV


def _ring_body(p_ref, out_ref, comm_ref, rs_send_sems, rs_recv_sems,
               ag_send_sems, ag_recv_sems, credit_sem):
    my = lax.axis_index("i")
    left = lax.rem(my - 1 + N_DEV, N_DEV)
    right = lax.rem(my + 1, N_DEV)

    barrier_sem = pltpu.get_barrier_semaphore()
    for nbr in (left, right):
        pl.semaphore_signal(barrier_sem, inc=1, device_id=(nbr,),
                            device_id_type=pl.DeviceIdType.MESH)
    pl.semaphore_wait(barrier_sem, 2)

    out_ref[...] = p_ref[...]

    for s in range(N_DEV - 1):
        slot = s % 2
        c_send = lax.rem(my - s + N_DEV, N_DEV)
        c_recv = lax.rem(my - s - 1 + 2 * N_DEV, N_DEV)
        if s >= 2:
            pl.semaphore_wait(credit_sem, 1)
        rdma = pltpu.make_async_remote_copy(
            src_ref=out_ref.at[pl.ds(c_send * CH, CH), :],
            dst_ref=comm_ref.at[slot],
            send_sem=rs_send_sems.at[s],
            recv_sem=rs_recv_sems.at[s],
            device_id=(right,),
            device_id_type=pl.DeviceIdType.MESH,
        )
        rdma.start()
        rdma.wait()
        out_ref[pl.ds(c_recv * CH, CH), :] += comm_ref[slot, :, :]
        if s < N_DEV - 3:
            pl.semaphore_signal(credit_sem, inc=1, device_id=(left,),
                                device_id_type=pl.DeviceIdType.MESH)

    for s in range(N_DEV - 1):
        c = lax.rem(my + 1 - s + 2 * N_DEV, N_DEV)
        rdma = pltpu.make_async_remote_copy(
            src_ref=out_ref.at[pl.ds(c * CH, CH), :],
            dst_ref=out_ref.at[pl.ds(c * CH, CH), :],
            send_sem=ag_send_sems.at[s],
            recv_sem=ag_recv_sems.at[s],
            device_id=(right,),
            device_id_type=pl.DeviceIdType.MESH,
        )
        rdma.start()
        rdma.wait()


def kernel(x, Wq, K_ext, V_ext, Wo):
    i = lax.axis_index("i")

    x2 = x.reshape(SQ, DM).astype(jnp.bfloat16)
    K = K_ext.reshape(SKV, HQ_LOCAL, DH).transpose(1, 0, 2)
    V = V_ext.reshape(SKV, HQ_LOCAL, DH).transpose(1, 0, 2)
    Wq_i = lax.dynamic_slice_in_dim(Wq, i * (HQ_LOCAL * DH), HQ_LOCAL * DH, 1)
    Wo_i = lax.dynamic_slice_in_dim(Wo, i * (HQ_LOCAL * DH), HQ_LOCAL * DH, 0)

    partial = pl.pallas_call(
        _attn_body,
        grid=(NQ, HQ_LOCAL),
        in_specs=[
            pl.BlockSpec((QBLK, DM), lambda qi, h: (qi, 0)),
            pl.BlockSpec((DM, DH), lambda qi, h: (0, h)),
            pl.BlockSpec((1, SKV, DH), lambda qi, h: (h, 0, 0)),
            pl.BlockSpec((1, SKV, DH), lambda qi, h: (h, 0, 0)),
            pl.BlockSpec((DH, DM), lambda qi, h: (h, 0)),
        ],
        out_specs=pl.BlockSpec((QBLK, DM), lambda qi, h: (qi, 0)),
        out_shape=jax.ShapeDtypeStruct((SQ, DM), jnp.float32),
    )(x2, Wq_i, K, V, Wo_i)

    out = pl.pallas_call(
        _ring_body,
        out_shape=jax.ShapeDtypeStruct((SQ, DM), jnp.float32),
        in_specs=[pl.BlockSpec(memory_space=pltpu.VMEM)],
        out_specs=pl.BlockSpec(memory_space=pltpu.VMEM),
        scratch_shapes=[
            pltpu.VMEM((2, CH, DM), jnp.float32),
            pltpu.SemaphoreType.DMA((N_DEV - 1,)),
            pltpu.SemaphoreType.DMA((N_DEV - 1,)),
            pltpu.SemaphoreType.DMA((N_DEV - 1,)),
            pltpu.SemaphoreType.DMA((N_DEV - 1,)),
            pltpu.SemaphoreType.REGULAR,
        ],
        compiler_params=pltpu.CompilerParams(collective_id=0),
    )(partial)

    return out.reshape(1, SQ, DM)


# device time: 274423 ns/iter; 2.9824x vs baseline; 1.2493x over previous
import jax
import jax.numpy as jnp
from jax import lax
from jax.experimental import pallas as pl
from jax.experimental.pallas import tpu as pltpu

N_DEV = 8
SQ = 2048
SKV = 2048
HQ_LOCAL = 8
DH = 128
DM = 1024
SCALE = 0.08838834764831843
QBLK = 512
NQ = SQ // QBLK
BLK = 64


def _attn_body(x_ref, wq_ref, k_ref, v_ref, wo_ref, out_ref):
    qi = pl.program_id(0)
    h = pl.program_id(1)

    xb = x_ref[...]
    wq = wq_ref[...].astype(jnp.bfloat16)
    q = lax.dot_general(xb, wq, (((1,), (0,)), ((), ())),
                        preferred_element_type=jnp.float32)
    qs = (q * SCALE).astype(jnp.bfloat16)

    kb = k_ref[...][0, :, :].astype(jnp.bfloat16)
    s = lax.dot_general(qs, kb, (((1,), (1,)), ((), ())),
                        preferred_element_type=jnp.float32)

    rowb = (qi * QBLK + lax.broadcasted_iota(jnp.int32, (QBLK, SKV), 0)) // BLK
    colb = lax.broadcasted_iota(jnp.int32, (QBLK, SKV), 1) // BLK
    s = jnp.where(colb <= rowb, s, -1e9)

    m = jnp.max(s, axis=1, keepdims=True)
    w = jnp.exp(s - m)
    denom = jnp.sum(w, axis=1, keepdims=True)
    wb = (w / denom).astype(jnp.bfloat16)

    vb = v_ref[...][0, :, :].astype(jnp.bfloat16)
    ctx = lax.dot_general(wb, vb, (((1,), (0,)), ((), ())),
                          preferred_element_type=jnp.float32)
    ctxb = ctx.astype(jnp.bfloat16)

    wo = wo_ref[...].astype(jnp.bfloat16)
    p = lax.dot_general(ctxb, wo, (((1,), (0,)), ((), ())),
                        preferred_element_type=jnp.float32)

    @pl.when(h == 0)
    def _():
        out_ref[...] = p

    @pl.when(h != 0)
    def _():
        out_ref[...] += p


CH = SQ // N_DEV


def _ring_body(p_ref, out_ref, send_buf, recv_buf, ag_buf,
               rs_send_sems, rs_recv_sems, ag_send_sems, ag_recv_sems,
               credit_sem):
    my = lax.axis_index("i")
    left = lax.rem(my - 1 + N_DEV, N_DEV)
    right = lax.rem(my + 1, N_DEV)

    barrier_sem = pltpu.get_barrier_semaphore()
    for nbr in (left, right):
        pl.semaphore_signal(barrier_sem, inc=1, device_id=(nbr,),
                            device_id_type=pl.DeviceIdType.MESH)
    pl.semaphore_wait(barrier_sem, 2)

    out_ref[...] = p_ref[...]

    for s in range(N_DEV - 1):
        slot = s % 2
        c_send = lax.rem(my - s + N_DEV, N_DEV)
        c_recv = lax.rem(my - s - 1 + 2 * N_DEV, N_DEV)
        send_buf[slot, :, :] = out_ref[pl.ds(c_send * CH, CH), :].astype(
            jnp.bfloat16)
        if s >= 2:
            pl.semaphore_wait(credit_sem, 1)
        rdma = pltpu.make_async_remote_copy(
            src_ref=send_buf.at[slot],
            dst_ref=recv_buf.at[slot],
            send_sem=rs_send_sems.at[s],
            recv_sem=rs_recv_sems.at[s],
            device_id=(right,),
            device_id_type=pl.DeviceIdType.MESH,
        )
        rdma.start()
        rdma.wait()
        out_ref[pl.ds(c_recv * CH, CH), :] += recv_buf[slot, :, :].astype(
            jnp.float32)
        if s < N_DEV - 3:
            pl.semaphore_signal(credit_sem, inc=1, device_id=(left,),
                                device_id_type=pl.DeviceIdType.MESH)

    ag_buf[0, :, :] = out_ref[pl.ds(lax.rem(my + 1, N_DEV) * CH, CH), :].astype(
        jnp.bfloat16)
    for s in range(N_DEV - 1):
        send_slot = s % 2
        recv_slot = (s + 1) % 2
        c_recv = lax.rem(my - s + 2 * N_DEV, N_DEV)
        if s >= 1:
            pl.semaphore_wait(credit_sem, 1)
        rdma = pltpu.make_async_remote_copy(
            src_ref=ag_buf.at[send_slot],
            dst_ref=ag_buf.at[recv_slot],
            send_sem=ag_send_sems.at[s],
            recv_sem=ag_recv_sems.at[s],
            device_id=(right,),
            device_id_type=pl.DeviceIdType.MESH,
        )
        rdma.start()
        rdma.wait()
        out_ref[pl.ds(c_recv * CH, CH), :] = ag_buf[recv_slot, :, :].astype(
            jnp.float32)
        if s < N_DEV - 2:
            pl.semaphore_signal(credit_sem, inc=1, device_id=(left,),
                                device_id_type=pl.DeviceIdType.MESH)


def kernel(x, Wq, K_ext, V_ext, Wo):
    i = lax.axis_index("i")

    x2 = x.reshape(SQ, DM).astype(jnp.bfloat16)
    K = K_ext.reshape(SKV, HQ_LOCAL, DH).transpose(1, 0, 2)
    V = V_ext.reshape(SKV, HQ_LOCAL, DH).transpose(1, 0, 2)
    Wq_i = lax.dynamic_slice_in_dim(Wq, i * (HQ_LOCAL * DH), HQ_LOCAL * DH, 1)
    Wo_i = lax.dynamic_slice_in_dim(Wo, i * (HQ_LOCAL * DH), HQ_LOCAL * DH, 0)

    partial = pl.pallas_call(
        _attn_body,
        grid=(NQ, HQ_LOCAL),
        in_specs=[
            pl.BlockSpec((QBLK, DM), lambda qi, h: (qi, 0)),
            pl.BlockSpec((DM, DH), lambda qi, h: (0, h)),
            pl.BlockSpec((1, SKV, DH), lambda qi, h: (h, 0, 0)),
            pl.BlockSpec((1, SKV, DH), lambda qi, h: (h, 0, 0)),
            pl.BlockSpec((DH, DM), lambda qi, h: (h, 0)),
        ],
        out_specs=pl.BlockSpec((QBLK, DM), lambda qi, h: (qi, 0)),
        out_shape=jax.ShapeDtypeStruct((SQ, DM), jnp.float32),
    )(x2, Wq_i, K, V, Wo_i)

    out = pl.pallas_call(
        _ring_body,
        out_shape=jax.ShapeDtypeStruct((SQ, DM), jnp.float32),
        in_specs=[pl.BlockSpec(memory_space=pltpu.VMEM)],
        out_specs=pl.BlockSpec(memory_space=pltpu.VMEM),
        scratch_shapes=[
            pltpu.VMEM((2, CH, DM), jnp.bfloat16),
            pltpu.VMEM((2, CH, DM), jnp.bfloat16),
            pltpu.VMEM((2, CH, DM), jnp.bfloat16),
            pltpu.SemaphoreType.DMA((N_DEV - 1,)),
            pltpu.SemaphoreType.DMA((N_DEV - 1,)),
            pltpu.SemaphoreType.DMA((N_DEV - 1,)),
            pltpu.SemaphoreType.DMA((N_DEV - 1,)),
            pltpu.SemaphoreType.REGULAR,
        ],
        compiler_params=pltpu.CompilerParams(collective_id=0),
    )(partial)

    return out.reshape(1, SQ, DM)


# device time: 209529 ns/iter; 3.9060x vs baseline; 1.3097x over previous
import jax
import jax.numpy as jnp
from jax import lax
from jax.experimental import pallas as pl
from jax.experimental.pallas import tpu as pltpu

N_DEV = 8
SQ = 2048
SKV = 2048
HQ_LOCAL = 8
DH = 128
DM = 1024
SCALE = 0.08838834764831843
QBLK = 512
NQ = SQ // QBLK
BLK = 64


def _attn_body(x_ref, wq_ref, k_ref, v_ref, wo_ref, out_ref):
    h = pl.program_id(0)

    xb = x_ref[...]
    wq = wq_ref[...].astype(jnp.bfloat16)
    q = lax.dot_general(xb, wq, (((1,), (0,)), ((), ())),
                        preferred_element_type=jnp.float32)
    qs = (q * SCALE).astype(jnp.bfloat16)

    kb = k_ref[...][0, :, :].astype(jnp.bfloat16)
    vb = v_ref[...][0, :, :].astype(jnp.bfloat16)

    ctx_parts = []
    for qi in range(NQ):
        kend = (qi + 1) * QBLK
        q_t = qs[qi * QBLK:(qi + 1) * QBLK, :]
        s = lax.dot_general(q_t, kb[:kend, :], (((1,), (1,)), ((), ())),
                            preferred_element_type=jnp.float32)
        row = qi * QBLK + lax.broadcasted_iota(jnp.int32, (QBLK, 1), 0)
        thresh = (row // BLK + 1) * BLK
        col = lax.broadcasted_iota(jnp.int32, (QBLK, kend), 1)
        w = jnp.exp(jnp.where(col < thresh, s, -1e9))
        denom = jnp.sum(w, axis=1, keepdims=True)
        ctx_t = lax.dot_general(w.astype(jnp.bfloat16), vb[:kend, :],
                                (((1,), (0,)), ((), ())),
                                preferred_element_type=jnp.float32)
        ctx_parts.append(ctx_t * (1.0 / denom))
    ctx = jnp.concatenate(ctx_parts, axis=0)
    ctxb = ctx.astype(jnp.bfloat16)

    wo = wo_ref[...].astype(jnp.bfloat16)
    p = lax.dot_general(ctxb, wo, (((1,), (0,)), ((), ())),
                        preferred_element_type=jnp.float32)

    @pl.when(h == 0)
    def _():
        out_ref[...] = p

    @pl.when(h != 0)
    def _():
        out_ref[...] += p


CH = SQ // N_DEV


def _ring_body(p_ref, out_ref, send_buf, recv_buf, ag_buf,
               rs_send_sems, rs_recv_sems, ag_send_sems, ag_recv_sems,
               credit_sem):
    my = lax.axis_index("i")
    left = lax.rem(my - 1 + N_DEV, N_DEV)
    right = lax.rem(my + 1, N_DEV)

    barrier_sem = pltpu.get_barrier_semaphore()
    for nbr in (left, right):
        pl.semaphore_signal(barrier_sem, inc=1, device_id=(nbr,),
                            device_id_type=pl.DeviceIdType.MESH)
    pl.semaphore_wait(barrier_sem, 2)

    out_ref[...] = p_ref[...]

    for s in range(N_DEV - 1):
        slot = s % 2
        c_send = lax.rem(my - s + N_DEV, N_DEV)
        c_recv = lax.rem(my - s - 1 + 2 * N_DEV, N_DEV)
        send_buf[slot, :, :] = out_ref[pl.ds(c_send * CH, CH), :].astype(
            jnp.bfloat16)
        if s >= 2:
            pl.semaphore_wait(credit_sem, 1)
        rdma = pltpu.make_async_remote_copy(
            src_ref=send_buf.at[slot],
            dst_ref=recv_buf.at[slot],
            send_sem=rs_send_sems.at[s],
            recv_sem=rs_recv_sems.at[s],
            device_id=(right,),
            device_id_type=pl.DeviceIdType.MESH,
        )
        rdma.start()
        rdma.wait()
        out_ref[pl.ds(c_recv * CH, CH), :] += recv_buf[slot, :, :].astype(
            jnp.float32)
        if s < N_DEV - 3:
            pl.semaphore_signal(credit_sem, inc=1, device_id=(left,),
                                device_id_type=pl.DeviceIdType.MESH)

    ag_buf[0, :, :] = out_ref[pl.ds(lax.rem(my + 1, N_DEV) * CH, CH), :].astype(
        jnp.bfloat16)
    for s in range(N_DEV - 1):
        send_slot = s % 2
        recv_slot = (s + 1) % 2
        c_recv = lax.rem(my - s + 2 * N_DEV, N_DEV)
        if s >= 1:
            pl.semaphore_wait(credit_sem, 1)
        rdma = pltpu.make_async_remote_copy(
            src_ref=ag_buf.at[send_slot],
            dst_ref=ag_buf.at[recv_slot],
            send_sem=ag_send_sems.at[s],
            recv_sem=ag_recv_sems.at[s],
            device_id=(right,),
            device_id_type=pl.DeviceIdType.MESH,
        )
        rdma.start()
        rdma.wait()
        out_ref[pl.ds(c_recv * CH, CH), :] = ag_buf[recv_slot, :, :].astype(
            jnp.float32)
        if s < N_DEV - 2:
            pl.semaphore_signal(credit_sem, inc=1, device_id=(left,),
                                device_id_type=pl.DeviceIdType.MESH)


def kernel(x, Wq, K_ext, V_ext, Wo):
    i = lax.axis_index("i")

    x2 = x.reshape(SQ, DM).astype(jnp.bfloat16)
    K = K_ext.reshape(SKV, HQ_LOCAL, DH).transpose(1, 0, 2)
    V = V_ext.reshape(SKV, HQ_LOCAL, DH).transpose(1, 0, 2)
    Wq_i = lax.dynamic_slice_in_dim(Wq, i * (HQ_LOCAL * DH), HQ_LOCAL * DH, 1)
    Wo_i = lax.dynamic_slice_in_dim(Wo, i * (HQ_LOCAL * DH), HQ_LOCAL * DH, 0)

    partial = pl.pallas_call(
        _attn_body,
        grid=(HQ_LOCAL,),
        in_specs=[
            pl.BlockSpec((SQ, DM), lambda h: (0, 0)),
            pl.BlockSpec((DM, DH), lambda h: (0, h)),
            pl.BlockSpec((1, SKV, DH), lambda h: (h, 0, 0)),
            pl.BlockSpec((1, SKV, DH), lambda h: (h, 0, 0)),
            pl.BlockSpec((DH, DM), lambda h: (h, 0)),
        ],
        out_specs=pl.BlockSpec((SQ, DM), lambda h: (0, 0)),
        out_shape=jax.ShapeDtypeStruct((SQ, DM), jnp.float32),
    )(x2, Wq_i, K, V, Wo_i)

    out = pl.pallas_call(
        _ring_body,
        out_shape=jax.ShapeDtypeStruct((SQ, DM), jnp.float32),
        in_specs=[pl.BlockSpec(memory_space=pltpu.VMEM)],
        out_specs=pl.BlockSpec(memory_space=pltpu.VMEM),
        scratch_shapes=[
            pltpu.VMEM((2, CH, DM), jnp.bfloat16),
            pltpu.VMEM((2, CH, DM), jnp.bfloat16),
            pltpu.VMEM((2, CH, DM), jnp.bfloat16),
            pltpu.SemaphoreType.DMA((N_DEV - 1,)),
            pltpu.SemaphoreType.DMA((N_DEV - 1,)),
            pltpu.SemaphoreType.DMA((N_DEV - 1,)),
            pltpu.SemaphoreType.DMA((N_DEV - 1,)),
            pltpu.SemaphoreType.REGULAR,
        ],
        compiler_params=pltpu.CompilerParams(collective_id=0),
    )(partial)

    return out.reshape(1, SQ, DM)


# device time: 146263 ns/iter; 5.5956x vs baseline; 1.4325x over previous
import jax
import jax.numpy as jnp
from jax import lax
from jax.experimental import pallas as pl
from jax.experimental.pallas import tpu as pltpu

N_DEV = 8
SQ = 2048
SKV = 2048
HQ_LOCAL = 8
DH = 128
DM = 1024
SCALE = 0.08838834764831843
QBLK = 512
NQ = SQ // QBLK
BLK = 64


def _attn_body(x_ref, wq_ref, k_ref, v_ref, wo_ref, out_ref):
    h = pl.program_id(0)

    xb = x_ref[...]
    wq = wq_ref[...].astype(jnp.bfloat16)
    q = lax.dot_general(xb, wq, (((1,), (0,)), ((), ())),
                        preferred_element_type=jnp.float32)
    qs = (q * SCALE).astype(jnp.bfloat16)

    kb = k_ref[...][0, :, :].astype(jnp.bfloat16)
    vb = v_ref[...][0, :, :].astype(jnp.bfloat16)

    ctx_parts = []
    for qi in range(NQ):
        kend = (qi + 1) * QBLK
        q_t = qs[qi * QBLK:(qi + 1) * QBLK, :]
        s = lax.dot_general(q_t, kb[:kend, :], (((1,), (1,)), ((), ())),
                            preferred_element_type=jnp.float32)
        row = qi * QBLK + lax.broadcasted_iota(jnp.int32, (QBLK, 1), 0)
        thresh = (row // BLK + 1) * BLK
        col = lax.broadcasted_iota(jnp.int32, (QBLK, kend), 1)
        w = jnp.exp(jnp.where(col < thresh, s, -1e9))
        denom = jnp.sum(w, axis=1, keepdims=True)
        ctx_t = lax.dot_general(w.astype(jnp.bfloat16), vb[:kend, :],
                                (((1,), (0,)), ((), ())),
                                preferred_element_type=jnp.float32)
        ctx_parts.append(ctx_t * (1.0 / denom))
    ctx = jnp.concatenate(ctx_parts, axis=0)
    ctxb = ctx.astype(jnp.bfloat16)

    wo = wo_ref[...].astype(jnp.bfloat16)
    p = lax.dot_general(ctxb, wo, (((1,), (0,)), ((), ())),
                        preferred_element_type=jnp.float32)

    @pl.when(h == 0)
    def _():
        out_ref[...] = p

    @pl.when(h != 0)
    def _():
        out_ref[...] += p


HALF = SQ // 2


def _ring_body(p_ref, out_ref, sendA, sendB,
               rsA0, rsA1, rsA2, rsB0, rsB1, rsB2,
               agA0, agA1, agA2, agB0, agB1, agB2,
               rsA_send, rsA_recv, rsB_send, rsB_recv,
               agA_send, agA_recv, agB_send, agB_recv):
    my = lax.axis_index("i")
    q = my & 3
    xbit = (q ^ (q >> 1)) & 1
    ybit = q >> 1
    zbit = my >> 2
    p_x = my ^ 1
    p_y = (my & 4) | (3 - q)
    p_z = my ^ 4

    barrier_sem = pltpu.get_barrier_semaphore()
    for nbr in (p_x, p_y, p_z):
        pl.semaphore_signal(barrier_sem, inc=1, device_id=(nbr,),
                            device_id_type=pl.DeviceIdType.MESH)
    pl.semaphore_wait(barrier_sem, 3)

    out_ref[...] = p_ref[...]

    dimsA = [(p_z, zbit), (p_y, ybit), (p_x, xbit)]
    dimsB = [(p_x, xbit), (p_z, zbit), (p_y, ybit)]
    rsA = [rsA0, rsA1, rsA2]
    rsB = [rsB0, rsB1, rsB2]
    agA = [agA0, agA1, agA2]
    agB = [agB0, agB1, agB2]

    loA = my * 0
    loB = my * 0 + HALF
    half = HALF // 2
    for d in range(3):
        pA, bA = dimsA[d]
        pB, bB = dimsB[d]
        send_loA = loA + (1 - bA) * half
        send_loB = loB + (1 - bB) * half
        sendA[0:half, :] = out_ref[pl.ds(send_loA, half), :].astype(
            jnp.bfloat16)
        sendB[0:half, :] = out_ref[pl.ds(send_loB, half), :].astype(
            jnp.bfloat16)
        rdmaA = pltpu.make_async_remote_copy(
            src_ref=sendA.at[pl.ds(0, half), :], dst_ref=rsA[d],
            send_sem=rsA_send.at[d], recv_sem=rsA_recv.at[d],
            device_id=(pA,), device_id_type=pl.DeviceIdType.MESH)
        rdmaB = pltpu.make_async_remote_copy(
            src_ref=sendB.at[pl.ds(0, half), :], dst_ref=rsB[d],
            send_sem=rsB_send.at[d], recv_sem=rsB_recv.at[d],
            device_id=(pB,), device_id_type=pl.DeviceIdType.MESH)
        rdmaA.start()
        rdmaB.start()
        rdmaA.wait()
        rdmaB.wait()
        loA = loA + bA * half
        loB = loB + bB * half
        out_ref[pl.ds(loA, half), :] += rsA[d][...].astype(jnp.float32)
        out_ref[pl.ds(loB, half), :] += rsB[d][...].astype(jnp.float32)
        half //= 2

    length = HALF // 8
    for d in range(3):
        pA, bA = dimsA[2 - d]
        pB, bB = dimsB[2 - d]
        sendA[0:length, :] = out_ref[pl.ds(loA, length), :].astype(
            jnp.bfloat16)
        sendB[0:length, :] = out_ref[pl.ds(loB, length), :].astype(
            jnp.bfloat16)
        rdmaA = pltpu.make_async_remote_copy(
            src_ref=sendA.at[pl.ds(0, length), :], dst_ref=agA[d],
            send_sem=agA_send.at[d], recv_sem=agA_recv.at[d],
            device_id=(pA,), device_id_type=pl.DeviceIdType.MESH)
        rdmaB = pltpu.make_async_remote_copy(
            src_ref=sendB.at[pl.ds(0, length), :], dst_ref=agB[d],
            send_sem=agB_send.at[d], recv_sem=agB_recv.at[d],
            device_id=(pB,), device_id_type=pl.DeviceIdType.MESH)
        rdmaA.start()
        rdmaB.start()
        rdmaA.wait()
        rdmaB.wait()
        sib_loA = loA + (1 - 2 * bA) * length
        sib_loB = loB + (1 - 2 * bB) * length
        out_ref[pl.ds(sib_loA, length), :] = agA[d][...].astype(jnp.float32)
        out_ref[pl.ds(sib_loB, length), :] = agB[d][...].astype(jnp.float32)
        loA = loA - bA * length
        loB = loB - bB * length
        length *= 2


def kernel(x, Wq, K_ext, V_ext, Wo):
    i = lax.axis_index("i")

    x2 = x.reshape(SQ, DM).astype(jnp.bfloat16)
    K = K_ext.reshape(SKV, HQ_LOCAL, DH).transpose(1, 0, 2)
    V = V_ext.reshape(SKV, HQ_LOCAL, DH).transpose(1, 0, 2)
    Wq_i = lax.dynamic_slice_in_dim(Wq, i * (HQ_LOCAL * DH), HQ_LOCAL * DH, 1)
    Wo_i = lax.dynamic_slice_in_dim(Wo, i * (HQ_LOCAL * DH), HQ_LOCAL * DH, 0)

    partial = pl.pallas_call(
        _attn_body,
        grid=(HQ_LOCAL,),
        in_specs=[
            pl.BlockSpec((SQ, DM), lambda h: (0, 0)),
            pl.BlockSpec((DM, DH), lambda h: (0, h)),
            pl.BlockSpec((1, SKV, DH), lambda h: (h, 0, 0)),
            pl.BlockSpec((1, SKV, DH), lambda h: (h, 0, 0)),
            pl.BlockSpec((DH, DM), lambda h: (h, 0)),
        ],
        out_specs=pl.BlockSpec((SQ, DM), lambda h: (0, 0)),
        out_shape=jax.ShapeDtypeStruct((SQ, DM), jnp.float32),
    )(x2, Wq_i, K, V, Wo_i)

    out = pl.pallas_call(
        _ring_body,
        out_shape=jax.ShapeDtypeStruct((SQ, DM), jnp.float32),
        in_specs=[pl.BlockSpec(memory_space=pltpu.VMEM)],
        out_specs=pl.BlockSpec(memory_space=pltpu.VMEM),
        scratch_shapes=[
            pltpu.VMEM((HALF // 2, DM), jnp.bfloat16),
            pltpu.VMEM((HALF // 2, DM), jnp.bfloat16),
            pltpu.VMEM((HALF // 2, DM), jnp.bfloat16),
            pltpu.VMEM((HALF // 4, DM), jnp.bfloat16),
            pltpu.VMEM((HALF // 8, DM), jnp.bfloat16),
            pltpu.VMEM((HALF // 2, DM), jnp.bfloat16),
            pltpu.VMEM((HALF // 4, DM), jnp.bfloat16),
            pltpu.VMEM((HALF // 8, DM), jnp.bfloat16),
            pltpu.VMEM((HALF // 8, DM), jnp.bfloat16),
            pltpu.VMEM((HALF // 4, DM), jnp.bfloat16),
            pltpu.VMEM((HALF // 2, DM), jnp.bfloat16),
            pltpu.VMEM((HALF // 8, DM), jnp.bfloat16),
            pltpu.VMEM((HALF // 4, DM), jnp.bfloat16),
            pltpu.VMEM((HALF // 2, DM), jnp.bfloat16),
            pltpu.SemaphoreType.DMA((3,)),
            pltpu.SemaphoreType.DMA((3,)),
            pltpu.SemaphoreType.DMA((3,)),
            pltpu.SemaphoreType.DMA((3,)),
            pltpu.SemaphoreType.DMA((3,)),
            pltpu.SemaphoreType.DMA((3,)),
            pltpu.SemaphoreType.DMA((3,)),
            pltpu.SemaphoreType.DMA((3,)),
        ],
        compiler_params=pltpu.CompilerParams(collective_id=0),
    )(partial)

    return out.reshape(1, SQ, DM)


# device time: 140361 ns/iter; 5.8309x vs baseline; 1.0420x over previous
import jax
import jax.numpy as jnp
from jax import lax
from jax.experimental import pallas as pl
from jax.experimental.pallas import tpu as pltpu

N_DEV = 8
SQ = 2048
SKV = 2048
HQ_LOCAL = 8
DH = 128
DM = 1024
SCALE = 0.08838834764831843
QBLK = 512
NQ = SQ // QBLK
BLK = 64
HALF = SQ // 2


def _allreduce(out_ref, sendA, sendB, rsA, rsB, agA, agB,
               rsA_send, rsA_recv, rsB_send, rsB_recv,
               agA_send, agA_recv, agB_send, agB_recv):
    my = lax.axis_index("i")
    q = my & 3
    xbit = (q ^ (q >> 1)) & 1
    ybit = q >> 1
    zbit = my >> 2
    p_x = my ^ 1
    p_y = (my & 4) | (3 - q)
    p_z = my ^ 4

    barrier_sem = pltpu.get_barrier_semaphore()
    for nbr in (p_x, p_y, p_z):
        pl.semaphore_signal(barrier_sem, inc=1, device_id=(nbr,),
                            device_id_type=pl.DeviceIdType.MESH)
    pl.semaphore_wait(barrier_sem, 3)

    dimsA = [(p_z, zbit), (p_y, ybit), (p_x, xbit)]
    dimsB = [(p_x, xbit), (p_z, zbit), (p_y, ybit)]

    loA = my * 0
    loB = my * 0 + HALF
    half = HALF // 2
    for d in range(3):
        pA, bA = dimsA[d]
        pB, bB = dimsB[d]
        send_loA = loA + (1 - bA) * half
        send_loB = loB + (1 - bB) * half
        sendA[0:half, :] = out_ref[pl.ds(send_loA, half), :].astype(
            jnp.bfloat16)
        sendB[0:half, :] = out_ref[pl.ds(send_loB, half), :].astype(
            jnp.bfloat16)
        rdmaA = pltpu.make_async_remote_copy(
            src_ref=sendA.at[pl.ds(0, half), :], dst_ref=rsA[d],
            send_sem=rsA_send.at[d], recv_sem=rsA_recv.at[d],
            device_id=(pA,), device_id_type=pl.DeviceIdType.MESH)
        rdmaB = pltpu.make_async_remote_copy(
            src_ref=sendB.at[pl.ds(0, half), :], dst_ref=rsB[d],
            send_sem=rsB_send.at[d], recv_sem=rsB_recv.at[d],
            device_id=(pB,), device_id_type=pl.DeviceIdType.MESH)
        rdmaA.start()
        rdmaB.start()
        rdmaA.wait()
        rdmaB.wait()
        loA = loA + bA * half
        loB = loB + bB * half
        out_ref[pl.ds(loA, half), :] += rsA[d][...].astype(jnp.float32)
        out_ref[pl.ds(loB, half), :] += rsB[d][...].astype(jnp.float32)
        half //= 2

    length = HALF // 8
    for d in range(3):
        pA, bA = dimsA[2 - d]
        pB, bB = dimsB[2 - d]
        sendA[0:length, :] = out_ref[pl.ds(loA, length), :].astype(
            jnp.bfloat16)
        sendB[0:length, :] = out_ref[pl.ds(loB, length), :].astype(
            jnp.bfloat16)
        rdmaA = pltpu.make_async_remote_copy(
            src_ref=sendA.at[pl.ds(0, length), :], dst_ref=agA[d],
            send_sem=agA_send.at[d], recv_sem=agA_recv.at[d],
            device_id=(pA,), device_id_type=pl.DeviceIdType.MESH)
        rdmaB = pltpu.make_async_remote_copy(
            src_ref=sendB.at[pl.ds(0, length), :], dst_ref=agB[d],
            send_sem=agB_send.at[d], recv_sem=agB_recv.at[d],
            device_id=(pB,), device_id_type=pl.DeviceIdType.MESH)
        rdmaA.start()
        rdmaB.start()
        rdmaA.wait()
        rdmaB.wait()
        sib_loA = loA + (1 - 2 * bA) * length
        sib_loB = loB + (1 - 2 * bB) * length
        out_ref[pl.ds(sib_loA, length), :] = agA[d][...].astype(jnp.float32)
        out_ref[pl.ds(sib_loB, length), :] = agB[d][...].astype(jnp.float32)
        loA = loA - bA * length
        loB = loB - bB * length
        length *= 2


def _fused_body(idx_ref, x_ref, wq_ref, k_ref, v_ref, wo_ref, out_ref,
                sendA, sendB,
                rsA0, rsA1, rsA2, rsB0, rsB1, rsB2,
                agA0, agA1, agA2, agB0, agB1, agB2,
                rsA_send, rsA_recv, rsB_send, rsB_recv,
                agA_send, agA_recv, agB_send, agB_recv):
    h = pl.program_id(0)

    xb = x_ref[...]
    wq = wq_ref[...].astype(jnp.bfloat16)
    q = lax.dot_general(xb, wq, (((1,), (0,)), ((), ())),
                        preferred_element_type=jnp.float32)
    qs = (q * SCALE).astype(jnp.bfloat16)

    kb = k_ref[...][0, :, :]
    vb = v_ref[...][0, :, :]

    ctx_parts = []
    for qi in range(NQ):
        kend = (qi + 1) * QBLK
        q_t = qs[qi * QBLK:(qi + 1) * QBLK, :]
        s = lax.dot_general(q_t, kb[:kend, :], (((1,), (1,)), ((), ())),
                            preferred_element_type=jnp.float32)
        row = qi * QBLK + lax.broadcasted_iota(jnp.int32, (QBLK, 1), 0)
        thresh = (row // BLK + 1) * BLK
        col = lax.broadcasted_iota(jnp.int32, (QBLK, kend), 1)
        w = jnp.exp(jnp.where(col < thresh, s, -1e9))
        denom = jnp.sum(w, axis=1, keepdims=True)
        ctx_t = lax.dot_general(w.astype(jnp.bfloat16), vb[:kend, :],
                                (((1,), (0,)), ((), ())),
                                preferred_element_type=jnp.float32)
        ctx_parts.append(ctx_t * (1.0 / denom))
    ctx = jnp.concatenate(ctx_parts, axis=0)
    ctxb = ctx.astype(jnp.bfloat16)

    wo = wo_ref[...].astype(jnp.bfloat16)
    p = lax.dot_general(ctxb, wo, (((1,), (0,)), ((), ())),
                        preferred_element_type=jnp.float32)

    @pl.when(h == 0)
    def _():
        out_ref[...] = p

    @pl.when(h != 0)
    def _():
        out_ref[...] += p

    @pl.when(h == HQ_LOCAL - 1)
    def _():
        _allreduce(out_ref, sendA, sendB,
                   [rsA0, rsA1, rsA2], [rsB0, rsB1, rsB2],
                   [agA0, agA1, agA2], [agB0, agB1, agB2],
                   rsA_send, rsA_recv, rsB_send, rsB_recv,
                   agA_send, agA_recv, agB_send, agB_recv)


def kernel(x, Wq, K_ext, V_ext, Wo):
    i = lax.axis_index("i")
    idx = jnp.full((1,), i * HQ_LOCAL, dtype=jnp.int32)

    x2 = x.reshape(SQ, DM).astype(jnp.bfloat16)
    K = K_ext.reshape(SKV, HQ_LOCAL, DH).transpose(1, 0, 2).astype(
        jnp.bfloat16)
    V = V_ext.reshape(SKV, HQ_LOCAL, DH).transpose(1, 0, 2).astype(
        jnp.bfloat16)

    grid_spec = pltpu.PrefetchScalarGridSpec(
        num_scalar_prefetch=1,
        grid=(HQ_LOCAL,),
        in_specs=[
            pl.BlockSpec((SQ, DM), lambda h, idx: (0, 0)),
            pl.BlockSpec((DM, DH), lambda h, idx: (0, idx[0] + h)),
            pl.BlockSpec((1, SKV, DH), lambda h, idx: (h, 0, 0)),
            pl.BlockSpec((1, SKV, DH), lambda h, idx: (h, 0, 0)),
            pl.BlockSpec((DH, DM), lambda h, idx: (idx[0] + h, 0)),
        ],
        out_specs=pl.BlockSpec((SQ, DM), lambda h, idx: (0, 0)),
        scratch_shapes=[
            pltpu.VMEM((HALF // 2, DM), jnp.bfloat16),
            pltpu.VMEM((HALF // 2, DM), jnp.bfloat16),
            pltpu.VMEM((HALF // 2, DM), jnp.bfloat16),
            pltpu.VMEM((HALF // 4, DM), jnp.bfloat16),
            pltpu.VMEM((HALF // 8, DM), jnp.bfloat16),
            pltpu.VMEM((HALF // 2, DM), jnp.bfloat16),
            pltpu.VMEM((HALF // 4, DM), jnp.bfloat16),
            pltpu.VMEM((HALF // 8, DM), jnp.bfloat16),
            pltpu.VMEM((HALF // 8, DM), jnp.bfloat16),
            pltpu.VMEM((HALF // 4, DM), jnp.bfloat16),
            pltpu.VMEM((HALF // 2, DM), jnp.bfloat16),
            pltpu.VMEM((HALF // 8, DM), jnp.bfloat16),
            pltpu.VMEM((HALF // 4, DM), jnp.bfloat16),
            pltpu.VMEM((HALF // 2, DM), jnp.bfloat16),
            pltpu.SemaphoreType.DMA((3,)),
            pltpu.SemaphoreType.DMA((3,)),
            pltpu.SemaphoreType.DMA((3,)),
            pltpu.SemaphoreType.DMA((3,)),
            pltpu.SemaphoreType.DMA((3,)),
            pltpu.SemaphoreType.DMA((3,)),
            pltpu.SemaphoreType.DMA((3,)),
            pltpu.SemaphoreType.DMA((3,)),
        ],
    )

    out = pl.pallas_call(
        _fused_body,
        grid_spec=grid_spec,
        out_shape=jax.ShapeDtypeStruct((SQ, DM), jnp.float32),
        compiler_params=pltpu.CompilerParams(collective_id=0),
    )(idx, x2, Wq, K, V, Wo)

    return out.reshape(1, SQ, DM)


# device time: 136734 ns/iter; 5.9856x vs baseline; 1.0265x over previous
import jax
import jax.numpy as jnp
from jax import lax
from jax.experimental import pallas as pl
from jax.experimental.pallas import tpu as pltpu

N_DEV = 8
SQ = 2048
SKV = 2048
HQ_LOCAL = 8
DH = 128
DM = 1024
SCALE = 0.08838834764831843
QBLK = 512
NQ = SQ // QBLK
BLK = 64
GROUP = QBLK
SIZES = [GROUP // 2, GROUP // 4, GROUP // 8,
         GROUP // 8, GROUP // 4, GROUP // 2]


class _GroupAR:

    def __init__(self, base, dims6, out_ref, send_buf, step_bufs,
                 rs_send, rs_recv, ag_send, ag_recv, g):
        self.base = base
        self.dims = dims6
        self.out = out_ref
        self.send = send_buf
        self.bufs = step_bufs
        self.sems = (rs_send, rs_recv, ag_send, ag_recv)
        self.g = g
        self.lo = base
        self.next_d = 0
        self.pending = None

    def issue(self):
        d = self.next_d
        self.next_d += 1
        p, b = self.dims[d]
        n = SIZES[d]
        if d < 3:
            src_lo = self.lo + (1 - b) * n
            ssem = self.sems[0].at[self.g, d]
            rsem = self.sems[1].at[self.g, d]
        else:
            src_lo = self.lo
            ssem = self.sems[2].at[self.g, d - 3]
            rsem = self.sems[3].at[self.g, d - 3]
        self.send[0:n, :] = self.out[pl.ds(src_lo, n), :].astype(jnp.bfloat16)
        rdma = pltpu.make_async_remote_copy(
            src_ref=self.send.at[pl.ds(0, n), :],
            dst_ref=self.bufs[d],
            send_sem=ssem, recv_sem=rsem,
            device_id=(p,), device_id_type=pl.DeviceIdType.MESH)
        rdma.start()
        self.pending = (rdma, d)

    def complete(self):
        rdma, d = self.pending
        self.pending = None
        p, b = self.dims[d]
        n = SIZES[d]
        rdma.wait()
        if d < 3:
            self.lo = self.lo + b * n
            self.out[pl.ds(self.lo, n), :] += self.bufs[d][...].astype(
                jnp.float32)
        else:
            sib = self.lo + (1 - 2 * b) * n
            self.out[pl.ds(sib, n), :] = self.bufs[d][...].astype(jnp.float32)
            self.lo = self.lo - b * n

    def advance(self, nsteps):
        for _ in range(nsteps):
            if self.pending is None:
                return
            self.complete()
            if self.next_d < 6:
                self.issue()

    @property
    def done(self):
        return self.pending is None and self.next_d >= 6


def _body(x_ref, wq_ref, k_ref, v_ref, wo_ref, out_ref, *scr):
    my = lax.axis_index("i")
    q = my & 3
    xbit = (q ^ (q >> 1)) & 1
    ybit = q >> 1
    zbit = my >> 2
    p_x = my ^ 1
    p_y = (my & 4) | (3 - q)
    p_z = my ^ 4
    dim_of = {"x": (p_x, xbit), "y": (p_y, ybit), "z": (p_z, zbit)}
    orders = [["z", "y", "x"], ["x", "z", "y"],
              ["y", "x", "z"], ["z", "y", "x"]]

    barrier_sem = pltpu.get_barrier_semaphore()
    for nbr in (p_x, p_y, p_z):
        pl.semaphore_signal(barrier_sem, inc=1, device_id=(nbr,),
                            device_id_type=pl.DeviceIdType.MESH)
    pl.semaphore_wait(barrier_sem, 3)

    rs_send, rs_recv, ag_send, ag_recv = scr[28:32]
    groups = []
    for g in range(NQ):
        rs_order = orders[g]
        dims6 = ([dim_of[c] for c in rs_order]
                 + [dim_of[c] for c in reversed(rs_order)])
        groups.append(_GroupAR(
            my * 0 + g * GROUP, dims6, out_ref,
            scr[g * 7], list(scr[g * 7 + 1:g * 7 + 7]),
            rs_send, rs_recv, ag_send, ag_recv, g))

    xb = x_ref[...]
    wq = wq_ref[...]
    wo = wo_ref[...]

    for g in range(NQ):
        kend = (g + 1) * QBLK
        x_t = xb[g * QBLK:(g + 1) * QBLK, :]
        row = g * QBLK + lax.broadcasted_iota(jnp.int32, (QBLK, 1), 0)
        thresh = (row // BLK + 1) * BLK
        col = lax.broadcasted_iota(jnp.int32, (QBLK, kend), 1)
        keep = col < thresh
        ctx_parts = []
        for h in range(HQ_LOCAL):
            qh = lax.dot_general(x_t, wq[:, h * DH:(h + 1) * DH],
                                 (((1,), (0,)), ((), ())),
                                 preferred_element_type=jnp.float32)
            qs = (qh * SCALE).astype(jnp.bfloat16)
            kb = k_ref[h, 0:kend, :]
            vb = v_ref[h, 0:kend, :]
            s = lax.dot_general(qs, kb, (((1,), (1,)), ((), ())),
                                preferred_element_type=jnp.float32)
            w = jnp.exp(jnp.where(keep, s, -1e9))
            denom = jnp.sum(w, axis=1, keepdims=True)
            ctx_t = lax.dot_general(w.astype(jnp.bfloat16), vb,
                                    (((1,), (0,)), ((), ())),
                                    preferred_element_type=jnp.float32)
            ctx_parts.append((ctx_t * (1.0 / denom)).astype(jnp.bfloat16))
        ctx = jnp.concatenate(ctx_parts, axis=1)
        p_tile = lax.dot_general(ctx, wo, (((1,), (0,)), ((), ())),
                                 preferred_element_type=jnp.float32)
        out_ref[g * QBLK:(g + 1) * QBLK, :] = p_tile

        for g2 in range(g):
            groups[g2].advance(2)
        groups[g].issue()

    while not all(gr.done for gr in groups):
        for gr in groups:
            gr.advance(1)


def kernel(x, Wq, K_ext, V_ext, Wo):
    i = lax.axis_index("i")

    x2 = x.reshape(SQ, DM).astype(jnp.bfloat16)
    K = K_ext.reshape(SKV, HQ_LOCAL, DH).transpose(1, 0, 2).astype(
        jnp.bfloat16)
    V = V_ext.reshape(SKV, HQ_LOCAL, DH).transpose(1, 0, 2).astype(
        jnp.bfloat16)
    Wq_i = lax.dynamic_slice_in_dim(
        Wq, i * (HQ_LOCAL * DH), HQ_LOCAL * DH, 1).astype(jnp.bfloat16)
    Wo_i = lax.dynamic_slice_in_dim(
        Wo, i * (HQ_LOCAL * DH), HQ_LOCAL * DH, 0).astype(jnp.bfloat16)

    scratch = []
    for _ in range(NQ):
        scratch.append(pltpu.VMEM((GROUP // 2, DM), jnp.bfloat16))
        for n in SIZES:
            scratch.append(pltpu.VMEM((n, DM), jnp.bfloat16))
    scratch += [pltpu.SemaphoreType.DMA((NQ, 3))] * 4

    out = pl.pallas_call(
        _body,
        out_shape=jax.ShapeDtypeStruct((SQ, DM), jnp.float32),
        in_specs=[pl.BlockSpec(memory_space=pltpu.VMEM)] * 5,
        out_specs=pl.BlockSpec(memory_space=pltpu.VMEM),
        scratch_shapes=scratch,
        compiler_params=pltpu.CompilerParams(collective_id=0),
    )(x2, Wq_i, K, V, Wo_i)

    return out.reshape(1, SQ, DM)


# device time: 125915 ns/iter; 6.4999x vs baseline; 1.0859x over previous
import jax
import jax.numpy as jnp
from jax import lax
from jax.experimental import pallas as pl
from jax.experimental.pallas import tpu as pltpu

N_DEV = 8
SQ = 2048
SKV = 2048
HQ_LOCAL = 8
DH = 128
DM = 1024
SCALE = 0.08838834764831843
QBLK = 512
NQ = SQ // QBLK
BLK = 64
GROUPS = [(0, 512), (512, 512), (1024, 512), (1536, 256), (1792, 256)]
ORDERS = [["z", "y", "x"], ["x", "z", "y"], ["y", "x", "z"],
          ["z", "y", "x"], ["x", "z", "y"]]
TILE_GROUPS = {0: [0], 1: [1], 2: [2], 3: [3, 4]}


def _sizes(rows):
    return [rows // 2, rows // 4, rows // 8, rows // 8, rows // 4, rows // 2]


class _GroupAR:

    def __init__(self, base, rows, dims6, out_ref, send_buf, step_bufs,
                 rs_send, rs_recv, ag_send, ag_recv, g):
        self.base = base
        self.sizes = _sizes(rows)
        self.dims = dims6
        self.out = out_ref
        self.send = send_buf
        self.bufs = step_bufs
        self.sems = (rs_send, rs_recv, ag_send, ag_recv)
        self.g = g
        self.lo = base
        self.next_d = 0
        self.pending = None

    def issue(self):
        d = self.next_d
        self.next_d += 1
        p, b = self.dims[d]
        n = self.sizes[d]
        if d < 3:
            src_lo = self.lo + (1 - b) * n
            ssem = self.sems[0].at[self.g, d]
            rsem = self.sems[1].at[self.g, d]
        else:
            src_lo = self.lo
            ssem = self.sems[2].at[self.g, d - 3]
            rsem = self.sems[3].at[self.g, d - 3]
        self.send[0:n, :] = self.out[pl.ds(src_lo, n), :].astype(jnp.bfloat16)
        rdma = pltpu.make_async_remote_copy(
            src_ref=self.send.at[pl.ds(0, n), :],
            dst_ref=self.bufs[d],
            send_sem=ssem, recv_sem=rsem,
            device_id=(p,), device_id_type=pl.DeviceIdType.MESH)
        rdma.start()
        self.pending = (rdma, d)

    def complete(self):
        rdma, d = self.pending
        self.pending = None
        p, b = self.dims[d]
        n = self.sizes[d]
        rdma.wait()
        if d < 3:
            self.lo = self.lo + b * n
            self.out[pl.ds(self.lo, n), :] += self.bufs[d][...].astype(
                jnp.float32)
        else:
            sib = self.lo + (1 - 2 * b) * n
            self.out[pl.ds(sib, n), :] = self.bufs[d][...].astype(jnp.float32)
            self.lo = self.lo - b * n

    def advance(self, nsteps):
        for _ in range(nsteps):
            if self.pending is None:
                return
            self.complete()
            if self.next_d < 6:
                self.issue()

    @property
    def done(self):
        return self.pending is None and self.next_d >= 6


def _body(x_ref, wq_ref, k_ref, v_ref, wo_ref, out_ref, *scr):
    my = lax.axis_index("i")
    q = my & 3
    xbit = (q ^ (q >> 1)) & 1
    ybit = q >> 1
    zbit = my >> 2
    p_x = my ^ 1
    p_y = (my & 4) | (3 - q)
    p_z = my ^ 4
    dim_of = {"x": (p_x, xbit), "y": (p_y, ybit), "z": (p_z, zbit)}

    barrier_sem = pltpu.get_barrier_semaphore()
    for nbr in (p_x, p_y, p_z):
        pl.semaphore_signal(barrier_sem, inc=1, device_id=(nbr,),
                            device_id_type=pl.DeviceIdType.MESH)
    pl.semaphore_wait(barrier_sem, 3)

    ngroups = len(GROUPS)
    rs_send, rs_recv, ag_send, ag_recv = scr[7 * ngroups:7 * ngroups + 4]
    groups = []
    for g, (base, rows) in enumerate(GROUPS):
        rs_order = ORDERS[g]
        dims6 = ([dim_of[c] for c in rs_order]
                 + [dim_of[c] for c in reversed(rs_order)])
        groups.append(_GroupAR(
            my * 0 + base, rows, dims6, out_ref,
            scr[g * 7], list(scr[g * 7 + 1:g * 7 + 7]),
            rs_send, rs_recv, ag_send, ag_recv, g))

    xb = x_ref[...]
    wq = wq_ref[...]
    wo = wo_ref[...]

    for g in range(NQ):
        kend = (g + 1) * QBLK
        x_t = xb[g * QBLK:(g + 1) * QBLK, :]
        q_all = lax.dot_general(x_t, wq, (((1,), (0,)), ((), ())),
                                preferred_element_type=jnp.float32)
        qs_all = (q_all * SCALE).astype(jnp.bfloat16)
        row = g * QBLK + lax.broadcasted_iota(jnp.int32, (QBLK, 1), 0)
        thresh = (row // BLK + 1) * BLK
        col = lax.broadcasted_iota(jnp.int32, (QBLK, kend), 1)
        keep = col < thresh
        ctx_parts = []
        for h in range(HQ_LOCAL):
            qs = qs_all[:, h * DH:(h + 1) * DH]
            kb = k_ref[h, 0:kend, :]
            vb = v_ref[h, 0:kend, :]
            s = lax.dot_general(qs, kb, (((1,), (1,)), ((), ())),
                                preferred_element_type=jnp.float32)
            w = jnp.exp(jnp.where(keep, s, -1e9))
            denom = jnp.sum(w, axis=1, keepdims=True)
            ctx_t = lax.dot_general(w.astype(jnp.bfloat16), vb,
                                    (((1,), (0,)), ((), ())),
                                    preferred_element_type=jnp.float32)
            ctx_parts.append((ctx_t * (1.0 / denom)).astype(jnp.bfloat16))
            if h % 2 == 1:
                for g2 in range(g):
                    for gi in TILE_GROUPS[g2]:
                        groups[gi].advance(1)
        ctx = jnp.concatenate(ctx_parts, axis=1)
        p_tile = lax.dot_general(ctx, wo, (((1,), (0,)), ((), ())),
                                 preferred_element_type=jnp.float32)
        out_ref[g * QBLK:(g + 1) * QBLK, :] = p_tile

        for gi in TILE_GROUPS[g]:
            groups[gi].issue()

    while not all(gr.done for gr in groups):
        for gr in groups:
            gr.advance(1)


def kernel(x, Wq, K_ext, V_ext, Wo):
    i = lax.axis_index("i")

    x2 = x.reshape(SQ, DM).astype(jnp.bfloat16)
    K = K_ext.reshape(SKV, HQ_LOCAL, DH).transpose(1, 0, 2).astype(
        jnp.bfloat16)
    V = V_ext.reshape(SKV, HQ_LOCAL, DH).transpose(1, 0, 2).astype(
        jnp.bfloat16)
    Wq_i = lax.dynamic_slice_in_dim(
        Wq, i * (HQ_LOCAL * DH), HQ_LOCAL * DH, 1).astype(jnp.bfloat16)
    Wo_i = lax.dynamic_slice_in_dim(
        Wo, i * (HQ_LOCAL * DH), HQ_LOCAL * DH, 0).astype(jnp.bfloat16)

    scratch = []
    for _, rows in GROUPS:
        scratch.append(pltpu.VMEM((rows // 2, DM), jnp.bfloat16))
        for n in _sizes(rows):
            scratch.append(pltpu.VMEM((n, DM), jnp.bfloat16))
    scratch += [pltpu.SemaphoreType.DMA((len(GROUPS), 3))] * 4

    out = pl.pallas_call(
        _body,
        out_shape=jax.ShapeDtypeStruct((SQ, DM), jnp.float32),
        in_specs=[pl.BlockSpec(memory_space=pltpu.VMEM)] * 5,
        out_specs=pl.BlockSpec(memory_space=pltpu.VMEM),
        scratch_shapes=scratch,
        compiler_params=pltpu.CompilerParams(collective_id=0),
    )(x2, Wq_i, K, V, Wo_i)

    return out.reshape(1, SQ, DM)
